# Initial kernel scaffold; baseline (speedup 1.0000x reference)
import functools

import jax
import jax.numpy as jnp
from jax import lax
from jax.experimental import pallas as pl
from jax.experimental.pallas import tpu as pltpu
from jax.experimental.pallas import tpu_sc as plsc

N = 16384
B = 1024
NNZ = 268435

NC = 2
NS = 16
BC = 64
NBC = B // BC
G = 128
NG = -(-NNZ // (NS * G))
TILE_NNZ = NG * G
NNZ_PAD = NS * TILE_NNZ


def _sc_body(xTr, colg, rowg, valsg, out,
             col_ref, row_ref, vals_ref, gbuf, zbuf, slab, sem_g, sem_s):
  c = lax.axis_index("c")
  s = lax.axis_index("s")

  pltpu.sync_copy(colg.at[s], col_ref)
  pltpu.sync_copy(rowg.at[s], row_ref)
  pltpu.sync_copy(valsg.at[s], vals_ref)

  @pl.loop(0, 256)
  def _(r):
    for q in range(4):
      zbuf[r, pl.ds(q * 16, 16)] = jnp.zeros((16,), jnp.float32)

  base = s * (N // NS)

  @pl.loop(0, NBC // NC)
  def _(p):
    bc = c * (NBC // NC) + p

    for i in range(4):
      pltpu.sync_copy(zbuf, slab.at[pl.ds(base + i * 256, 256)])
    plsc.subcore_barrier()

    @pl.loop(0, NG)
    def _(g):
      pltpu.async_copy(xTr.at[bc].at[col_ref.at[g]], gbuf, sem_g).wait()

      @pl.loop(0, G)
      def _(j):
        v = vals_ref[g, j]
        for q in range(4):
          sl = pl.ds(q * 16, 16)
          gbuf[j, sl] = gbuf[j, sl] * v

      pltpu.async_copy(gbuf, slab.at[row_ref.at[g]], sem_s, add=True).wait()

    plsc.subcore_barrier()
    pltpu.sync_copy(slab.at[pl.ds(base, N // NS)],
                    out.at[bc, pl.ds(base, N // NS)])
    plsc.subcore_barrier()


@jax.jit
def kernel(x_affine, W_vals, W_row, W_col):
  xTr = x_affine.T.reshape(N, NBC, BC).transpose(1, 0, 2)

  pad = NNZ_PAD - NNZ
  colg = jnp.pad(W_col.astype(jnp.int32), (0, pad)).reshape(NS, NG, G)
  rowg = jnp.pad(W_row.astype(jnp.int32), (0, pad)).reshape(NS, NG, G)
  valsg = jnp.pad(W_vals, (0, pad)).reshape(NS, NG, G)

  mesh = plsc.VectorSubcoreMesh(core_axis_name="c", subcore_axis_name="s")
  sc = pl.kernel(
      _sc_body,
      out_type=jax.ShapeDtypeStruct((NBC, N, BC), jnp.float32),
      mesh=mesh,
      scratch_types=[
          pltpu.VMEM((NG, G), jnp.int32),
          pltpu.VMEM((NG, G), jnp.int32),
          pltpu.VMEM((NG, G), jnp.float32),
          pltpu.VMEM((G, BC), jnp.float32),
          pltpu.VMEM((256, BC), jnp.float32),
          pltpu.VMEM_SHARED((N, BC), jnp.float32),
          pltpu.SemaphoreType.DMA,
          pltpu.SemaphoreType.DMA,
      ],
  )
  outr = sc(xTr, colg, rowg, valsg)
  return outr.transpose(0, 2, 1).reshape(B, N)


# speedup vs baseline: 1.3035x; 1.3035x over previous
import functools

import jax
import jax.numpy as jnp
from jax import lax
from jax.experimental import pallas as pl
from jax.experimental.pallas import tpu as pltpu
from jax.experimental.pallas import tpu_sc as plsc

N = 16384
B = 1024
NNZ = 268435

NC = 2
NS = 16
BC = 64
NBC = B // BC
G = 128
NG = -(-NNZ // (NS * G))
TILE_NNZ = NG * G
NNZ_PAD = NS * TILE_NNZ


def _sc_body(xTr, colg, rowg, valsg, out,
             colb, rowb, valb, gbuf, zbuf, slab, sem_g, sem_s):
  c = lax.axis_index("c")
  s = lax.axis_index("s")

  @pl.loop(0, G)
  def _(r):
    for q in range(BC // 16):
      zbuf[r, pl.ds(q * 16, 16)] = jnp.zeros((16,), jnp.float32)

  base = s * (N // NS)

  @pl.loop(0, NBC // NC)
  def _(p):
    bc = c * (NBC // NC) + p

    for i in range(N // NS // G):
      pltpu.sync_copy(zbuf, slab.at[pl.ds(base + i * G, G)])
    plsc.subcore_barrier()

    @pl.loop(0, NG)
    def _(g):
      pltpu.sync_copy(colg.at[s, g], colb.at[0])
      pltpu.sync_copy(rowg.at[s, g], rowb.at[0])
      pltpu.sync_copy(valsg.at[s, g], valb.at[0])

      pltpu.async_copy(xTr.at[bc].at[colb.at[0]], gbuf, sem_g).wait()

      @pl.loop(0, G // 16)
      def _(t):
        vv = valb[0, pl.ds(t * 16, 16)]
        for l in range(16):
          v = vv[l]
          j = t * 16 + l
          for q in range(BC // 16):
            sl = pl.ds(q * 16, 16)
            gbuf[j, sl] = gbuf[j, sl] * v

      pltpu.async_copy(gbuf, slab.at[rowb.at[0]], sem_s, add=True).wait()

    plsc.subcore_barrier()
    pltpu.sync_copy(slab.at[pl.ds(base, N // NS)],
                    out.at[bc, pl.ds(base, N // NS)])
    plsc.subcore_barrier()


@jax.jit
def kernel(x_affine, W_vals, W_row, W_col):
  xTr = x_affine.T.reshape(N, NBC, BC).transpose(1, 0, 2)

  pad = NNZ_PAD - NNZ
  colg = jnp.pad(W_col.astype(jnp.int32), (0, pad)).reshape(NS, NG, G)
  rowg = jnp.pad(W_row.astype(jnp.int32), (0, pad)).reshape(NS, NG, G)
  valsg = jnp.pad(W_vals, (0, pad)).reshape(NS, NG, G)

  mesh = plsc.VectorSubcoreMesh(core_axis_name="c", subcore_axis_name="s")
  sc = pl.kernel(
      _sc_body,
      out_type=jax.ShapeDtypeStruct((NBC, N, BC), jnp.float32),
      mesh=mesh,
      compiler_params=pltpu.CompilerParams(use_tc_tiling_on_sc=False),
      scratch_types=[
          pltpu.VMEM((1, G), jnp.int32),
          pltpu.VMEM((1, G), jnp.int32),
          pltpu.VMEM((1, G), jnp.float32),
          pltpu.VMEM((G, BC), jnp.float32),
          pltpu.VMEM((G, BC), jnp.float32),
          pltpu.VMEM_SHARED((N, BC), jnp.float32),
          pltpu.SemaphoreType.DMA,
          pltpu.SemaphoreType.DMA,
      ],
  )
  outr = sc(xTr, colg, rowg, valsg)
  return outr.transpose(0, 2, 1).reshape(B, N)
